# baseline (speedup 1.0000x reference)
import functools

import jax
import jax.numpy as jnp
import numpy as np
from jax import lax
from jax.experimental import pallas as pl
from jax.experimental.pallas import tpu as pltpu
from jax.experimental.pallas import tpu_sc as plsc

N = 10000
E = 320000
IN_DIM = 128
OUT_DIM = 64

NUM_WORKERS = 32
CHUNK = 128
CHUNKS_PER_WORKER = 80
REAL_CHUNKS = E // CHUNK
E_PAD = NUM_WORKERS * CHUNKS_PER_WORKER * CHUNK
M_ROWS = 10240
STRIPE = M_ROWS // 16
ZROWS = 64

ROW_BLOCK = 1000
E_BLOCK = E_PAD // 10

NBUF = 3
NFULL = CHUNKS_PER_WORKER // NBUF
TAIL = CHUNKS_PER_WORKER - NFULL * NBUF

_PERM = np.concatenate(
    [np.arange(0, 16), np.arange(32, 48),
     np.arange(16, 32), np.arange(48, 64)]).astype(np.int32)


def _tc_a_body(f_ref, ei_ref, wl_ref, wd_ref, nf_ref, z_ref, src_ref, dst_ref):
    nf = jax.lax.dot_general(
        f_ref[...], wl_ref[...], (((1,), (1,)), ((), ())),
        preferred_element_type=jnp.float32)
    nf_ref[...] = nf
    u1 = jnp.tanh(nf)
    zp = jax.lax.dot_general(
        u1, wd_ref[...], (((1,), (1,)), ((), ())),
        preferred_element_type=jnp.float32)
    ba = jax.lax.bitcast_convert_type(zp[:, :OUT_DIM // 2], jnp.int32)
    bb = jax.lax.bitcast_convert_type(zp[:, OUT_DIM // 2:], jnp.int32)
    ba = ba + jnp.int32(0x7FFF) + ((ba >> 16) & jnp.int32(1))
    bb = bb + jnp.int32(0x7FFF) + ((bb >> 16) & jnp.int32(1))
    z_ref[...] = ((ba >> 16) & jnp.int32(0xFFFF)) | (bb & jnp.int32(-65536))
    src_ref[...] = ei_ref[0, :]
    dst_ref[...] = ei_ref[1, :]


def _tc_b_body(nf_ref, s0_ref, s1_ref, out_ref):
    m = s0_ref[...] + s1_ref[...]
    out_ref[...] = jnp.tanh(nf_ref[...] + jnp.maximum(m, 0.0))


def _sc_body(z_hbm, src_hbm, dst_hbm,
             s0_hbm, s1_hbm,
             src_v, zb_v, rb0, rb1, rb2, rf_v, db0, db1, db2,
             z_spmem, acc_spmem,
             gs0, gs1, gs2, ds0, ds1, ds2):
    rows_bufs = (rb0, rb1, rb2)
    dst_bufs = (db0, db1, db2)
    gsems = (gs0, gs1, gs2)
    dsems = (ds0, ds1, ds2)

    cid = lax.axis_index("c")
    sid = lax.axis_index("s")
    wid = sid * 2 + cid
    base_chunk = wid * CHUNKS_PER_WORKER

    def zrow(i, carry):
        for j in range(OUT_DIM // 16):
            zb_v[i, pl.ds(j * 16, 16)] = jnp.zeros((16,), jnp.float32)
        return carry

    lax.fori_loop(0, ZROWS, zrow, 0, unroll=False)

    zero_copies = [
        pltpu.async_copy(
            zb_v, acc_spmem.at[pl.ds(sid * STRIPE + k * ZROWS, ZROWS)], gs0)
        for k in range(STRIPE // ZROWS)]
    src_copy = pltpu.async_copy(
        src_hbm.at[pl.ds(base_chunk * CHUNK, CHUNKS_PER_WORKER * CHUNK)],
        src_v, gs1)
    z_copy = pltpu.async_copy(
        z_hbm.at[pl.ds(sid * STRIPE, STRIPE)],
        z_spmem.at[pl.ds(sid * STRIPE, STRIPE)], gs2)
    for cp in zero_copies:
        cp.wait()
    src_copy.wait()
    z_copy.wait()

    plsc.subcore_barrier()

    def issue(c, b):
        @pl.when((c < CHUNKS_PER_WORKER) & (base_chunk + c < REAL_CHUNKS))
        def _():
            pltpu.async_copy(
                dst_hbm.at[pl.ds((base_chunk + c) * CHUNK, CHUNK)],
                dst_bufs[b], dsems[b])
            pltpu.async_copy(
                z_spmem.at[src_v.at[pl.ds(c * CHUNK, CHUNK)]],
                rows_bufs[b], gsems[b])

    def drain(c, b):
        @pl.when(base_chunk + c < REAL_CHUNKS)
        def _():
            pltpu.make_async_copy(
                dst_hbm.at[pl.ds((base_chunk + c) * CHUNK, CHUNK)],
                dst_bufs[b], dsems[b]).wait()
            pltpu.make_async_copy(
                z_spmem.at[src_v.at[pl.ds(c * CHUNK, CHUNK)]],
                rows_bufs[b], gsems[b]).wait()

            def widen(i, carry):
                for j in range(OUT_DIM // 32):
                    u = rows_bufs[b][i, pl.ds(16 * j, 16)]
                    lo = lax.bitcast_convert_type(u << jnp.int32(16),
                                                  jnp.float32)
                    hi = lax.bitcast_convert_type(u & jnp.int32(-65536),
                                                  jnp.float32)
                    rf_v[i, pl.ds(32 * j, 16)] = lo
                    rf_v[i, pl.ds(32 * j + 16, 16)] = hi
                return carry

            lax.fori_loop(0, CHUNK, widen, 0, unroll=8)
            pltpu.sync_copy(rf_v, acc_spmem.at[dst_bufs[b]], add=True)

    for b in range(NBUF):
        issue(b, b)

    def outer(g, carry):
        for b in range(NBUF):
            c = g * NBUF + b
            drain(c, b)
            issue(c + NBUF, b)
        return carry

    lax.fori_loop(0, NFULL, outer, 0, unroll=False)

    for t in range(TAIL):
        c = NFULL * NBUF + t
        drain(c, c % NBUF)

    plsc.subcore_barrier()

    @pl.when(cid == 0)
    def _():
        pltpu.sync_copy(acc_spmem.at[pl.ds(sid * STRIPE, STRIPE)],
                        s0_hbm.at[pl.ds(sid * STRIPE, STRIPE)])

    @pl.when(cid == 1)
    def _():
        pltpu.sync_copy(acc_spmem.at[pl.ds(sid * STRIPE, STRIPE)],
                        s1_hbm.at[pl.ds(sid * STRIPE, STRIPE)])


@jax.jit
def kernel(feature, edge_index, W_lin, W_dense):
    n_blocks = N // ROW_BLOCK

    nf, z, src, dst = pl.pallas_call(
        _tc_a_body,
        grid=(n_blocks,),
        in_specs=[
            pl.BlockSpec((ROW_BLOCK, IN_DIM), lambda r: (r, 0)),
            pl.BlockSpec((2, E_BLOCK), lambda r: (0, r)),
            pl.BlockSpec((OUT_DIM, IN_DIM), lambda r: (0, 0)),
            pl.BlockSpec((OUT_DIM, OUT_DIM), lambda r: (0, 0)),
        ],
        out_specs=[
            pl.BlockSpec((ROW_BLOCK, OUT_DIM), lambda r: (r, 0)),
            pl.BlockSpec((ROW_BLOCK, OUT_DIM // 2), lambda r: (r, 0)),
            pl.BlockSpec((E_BLOCK,), lambda r: (r,)),
            pl.BlockSpec((E_BLOCK,), lambda r: (r,)),
        ],
        out_shape=[
            jax.ShapeDtypeStruct((N, OUT_DIM), jnp.float32),
            jax.ShapeDtypeStruct((M_ROWS, OUT_DIM // 2), jnp.int32),
            jax.ShapeDtypeStruct((E_PAD,), jnp.int32),
            jax.ShapeDtypeStruct((E_PAD,), jnp.int32),
        ],
    )(feature, edge_index, W_lin, jnp.take(W_dense, _PERM, axis=0))

    sc_fn = pl.kernel(
        _sc_body,
        out_type=[
            jax.ShapeDtypeStruct((M_ROWS, OUT_DIM), jnp.float32),
            jax.ShapeDtypeStruct((M_ROWS, OUT_DIM), jnp.float32),
        ],
        mesh=plsc.VectorSubcoreMesh(core_axis_name="c", subcore_axis_name="s"),
        compiler_params=pltpu.CompilerParams(use_tc_tiling_on_sc=False),
        scratch_types=[
            pltpu.VMEM((CHUNKS_PER_WORKER * CHUNK,), jnp.int32),
            pltpu.VMEM((ZROWS, OUT_DIM), jnp.float32),
            pltpu.VMEM((CHUNK, OUT_DIM // 2), jnp.int32),
            pltpu.VMEM((CHUNK, OUT_DIM // 2), jnp.int32),
            pltpu.VMEM((CHUNK, OUT_DIM // 2), jnp.int32),
            pltpu.VMEM((CHUNK, OUT_DIM), jnp.float32),
            pltpu.VMEM((CHUNK,), jnp.int32),
            pltpu.VMEM((CHUNK,), jnp.int32),
            pltpu.VMEM((CHUNK,), jnp.int32),
            pltpu.VMEM_SHARED((M_ROWS, OUT_DIM // 2), jnp.int32),
            pltpu.VMEM_SHARED((M_ROWS, OUT_DIM), jnp.float32),
            pltpu.SemaphoreType.DMA,
            pltpu.SemaphoreType.DMA,
            pltpu.SemaphoreType.DMA,
            pltpu.SemaphoreType.DMA,
            pltpu.SemaphoreType.DMA,
            pltpu.SemaphoreType.DMA,
        ],
    )
    s0, s1 = sc_fn(z, src, dst)

    out = pl.pallas_call(
        _tc_b_body,
        grid=(n_blocks,),
        in_specs=[
            pl.BlockSpec((ROW_BLOCK, OUT_DIM), lambda r: (r, 0)),
            pl.BlockSpec((ROW_BLOCK, OUT_DIM), lambda r: (r, 0)),
            pl.BlockSpec((ROW_BLOCK, OUT_DIM), lambda r: (r, 0)),
        ],
        out_specs=pl.BlockSpec((ROW_BLOCK, OUT_DIM), lambda r: (r, 0)),
        out_shape=jax.ShapeDtypeStruct((N, OUT_DIM), jnp.float32),
    )(nf, s0, s1)
    return out


# speedup vs baseline: 1.4350x; 1.4350x over previous
import jax
import jax.numpy as jnp
from jax import lax
from jax.experimental import pallas as pl
from jax.experimental.pallas import tpu as pltpu
from jax.experimental.pallas import tpu_sc as plsc

N = 10000
E = 320000
IN_DIM = 128
OUT_DIM = 64

NUM_WORKERS = 32
CHUNK = 128
CHUNKS_PER_WORKER = 80
REAL_CHUNKS = E // CHUNK
E_PAD = NUM_WORKERS * CHUNKS_PER_WORKER * CHUNK
M_ROWS = 10240
STRIPE = M_ROWS // 16
ZROWS = 64

ROW_BLOCK = 1000
E_BLOCK = E_PAD // 10

NBUF = 3
NFULL = CHUNKS_PER_WORKER // NBUF
TAIL = CHUNKS_PER_WORKER - NFULL * NBUF


def _tc_a_body(f_ref, ei_ref, wl_ref, wd_ref, nf_ref, z_ref, src_ref, dst_ref):
    nf = jax.lax.dot_general(
        f_ref[...], wl_ref[...], (((1,), (1,)), ((), ())),
        preferred_element_type=jnp.float32)
    nf_ref[...] = nf
    u1 = jnp.tanh(nf)
    z_ref[...] = jax.lax.dot_general(
        u1, wd_ref[...], (((1,), (1,)), ((), ())),
        preferred_element_type=jnp.float32)
    src_ref[...] = ei_ref[0, :]
    dst_ref[...] = ei_ref[1, :]


def _tc_b_body(nf_ref, s0_ref, s1_ref, out_ref):
    m = s0_ref[...] + s1_ref[...]
    out_ref[...] = jnp.tanh(nf_ref[...] + jnp.maximum(m, 0.0))


def _sc_body(z_hbm, src_hbm, dst_hbm,
             s0_hbm, s1_hbm,
             src_v, zb_v, rb0, rb1, rb2, db0, db1, db2,
             z_spmem, acc_spmem,
             gs0, gs1, gs2, ds0, ds1, ds2):
    rows_bufs = (rb0, rb1, rb2)
    dst_bufs = (db0, db1, db2)
    gsems = (gs0, gs1, gs2)
    dsems = (ds0, ds1, ds2)

    cid = lax.axis_index("c")
    sid = lax.axis_index("s")
    wid = sid * 2 + cid
    base_chunk = wid * CHUNKS_PER_WORKER

    def zrow(i, carry):
        for j in range(OUT_DIM // 16):
            zb_v[i, pl.ds(j * 16, 16)] = jnp.zeros((16,), jnp.float32)
        return carry

    lax.fori_loop(0, ZROWS, zrow, 0, unroll=False)

    zero_copies = [
        pltpu.async_copy(
            zb_v, acc_spmem.at[pl.ds(sid * STRIPE + k * ZROWS, ZROWS)], gs0)
        for k in range(STRIPE // ZROWS)]
    src_copy = pltpu.async_copy(
        src_hbm.at[pl.ds(base_chunk * CHUNK, CHUNKS_PER_WORKER * CHUNK)],
        src_v, gs1)
    z_copy = pltpu.async_copy(
        z_hbm.at[pl.ds(sid * STRIPE, STRIPE)],
        z_spmem.at[pl.ds(sid * STRIPE, STRIPE)], gs2)
    for cp in zero_copies:
        cp.wait()
    src_copy.wait()
    z_copy.wait()

    plsc.subcore_barrier()

    def issue(c, b):
        @pl.when((c < CHUNKS_PER_WORKER) & (base_chunk + c < REAL_CHUNKS))
        def _():
            pltpu.async_copy(
                dst_hbm.at[pl.ds((base_chunk + c) * CHUNK, CHUNK)],
                dst_bufs[b], dsems[b])
            pltpu.async_copy(
                z_spmem.at[src_v.at[pl.ds(c * CHUNK, CHUNK)]],
                rows_bufs[b], gsems[b])

    def drain(c, b):
        @pl.when(base_chunk + c < REAL_CHUNKS)
        def _():
            pltpu.make_async_copy(
                dst_hbm.at[pl.ds((base_chunk + c) * CHUNK, CHUNK)],
                dst_bufs[b], dsems[b]).wait()
            pltpu.make_async_copy(
                z_spmem.at[src_v.at[pl.ds(c * CHUNK, CHUNK)]],
                rows_bufs[b], gsems[b]).wait()
            pltpu.sync_copy(rows_bufs[b], acc_spmem.at[dst_bufs[b]], add=True)

    for b in range(NBUF):
        issue(b, b)

    def outer(g, carry):
        for b in range(NBUF):
            c = g * NBUF + b
            drain(c, b)
            issue(c + NBUF, b)
        return carry

    lax.fori_loop(0, NFULL, outer, 0, unroll=False)

    for t in range(TAIL):
        c = NFULL * NBUF + t
        drain(c, c % NBUF)

    plsc.subcore_barrier()

    @pl.when(cid == 0)
    def _():
        pltpu.sync_copy(acc_spmem.at[pl.ds(sid * STRIPE, STRIPE)],
                        s0_hbm.at[pl.ds(sid * STRIPE, STRIPE)])

    @pl.when(cid == 1)
    def _():
        pltpu.sync_copy(acc_spmem.at[pl.ds(sid * STRIPE, STRIPE)],
                        s1_hbm.at[pl.ds(sid * STRIPE, STRIPE)])


@jax.jit
def kernel(feature, edge_index, W_lin, W_dense):
    n_blocks = N // ROW_BLOCK

    nf, z, src, dst = pl.pallas_call(
        _tc_a_body,
        grid=(n_blocks,),
        in_specs=[
            pl.BlockSpec((ROW_BLOCK, IN_DIM), lambda r: (r, 0)),
            pl.BlockSpec((2, E_BLOCK), lambda r: (0, r)),
            pl.BlockSpec((OUT_DIM, IN_DIM), lambda r: (0, 0)),
            pl.BlockSpec((OUT_DIM, OUT_DIM), lambda r: (0, 0)),
        ],
        out_specs=[
            pl.BlockSpec((ROW_BLOCK, OUT_DIM), lambda r: (r, 0)),
            pl.BlockSpec((ROW_BLOCK, OUT_DIM), lambda r: (r, 0)),
            pl.BlockSpec((E_BLOCK,), lambda r: (r,)),
            pl.BlockSpec((E_BLOCK,), lambda r: (r,)),
        ],
        out_shape=[
            jax.ShapeDtypeStruct((N, OUT_DIM), jnp.float32),
            jax.ShapeDtypeStruct((M_ROWS, OUT_DIM), jnp.float32),
            jax.ShapeDtypeStruct((E_PAD,), jnp.int32),
            jax.ShapeDtypeStruct((E_PAD,), jnp.int32),
        ],
    )(feature, edge_index, W_lin, W_dense)

    sc_fn = pl.kernel(
        _sc_body,
        out_type=[
            jax.ShapeDtypeStruct((M_ROWS, OUT_DIM), jnp.float32),
            jax.ShapeDtypeStruct((M_ROWS, OUT_DIM), jnp.float32),
        ],
        mesh=plsc.VectorSubcoreMesh(core_axis_name="c", subcore_axis_name="s"),
        compiler_params=pltpu.CompilerParams(use_tc_tiling_on_sc=False),
        scratch_types=[
            pltpu.VMEM((CHUNKS_PER_WORKER * CHUNK,), jnp.int32),
            pltpu.VMEM((ZROWS, OUT_DIM), jnp.float32),
            pltpu.VMEM((CHUNK, OUT_DIM), jnp.float32),
            pltpu.VMEM((CHUNK, OUT_DIM), jnp.float32),
            pltpu.VMEM((CHUNK, OUT_DIM), jnp.float32),
            pltpu.VMEM((CHUNK,), jnp.int32),
            pltpu.VMEM((CHUNK,), jnp.int32),
            pltpu.VMEM((CHUNK,), jnp.int32),
            pltpu.VMEM_SHARED((M_ROWS, OUT_DIM), jnp.float32),
            pltpu.VMEM_SHARED((M_ROWS, OUT_DIM), jnp.float32),
            pltpu.SemaphoreType.DMA,
            pltpu.SemaphoreType.DMA,
            pltpu.SemaphoreType.DMA,
            pltpu.SemaphoreType.DMA,
            pltpu.SemaphoreType.DMA,
            pltpu.SemaphoreType.DMA,
        ],
    )
    s0, s1 = sc_fn(z, src, dst)

    out = pl.pallas_call(
        _tc_b_body,
        grid=(n_blocks,),
        in_specs=[
            pl.BlockSpec((ROW_BLOCK, OUT_DIM), lambda r: (r, 0)),
            pl.BlockSpec((ROW_BLOCK, OUT_DIM), lambda r: (r, 0)),
            pl.BlockSpec((ROW_BLOCK, OUT_DIM), lambda r: (r, 0)),
        ],
        out_specs=pl.BlockSpec((ROW_BLOCK, OUT_DIM), lambda r: (r, 0)),
        out_shape=jax.ShapeDtypeStruct((N, OUT_DIM), jnp.float32),
    )(nf, s0, s1)
    return out
